# baseline (speedup 1.0000x reference)
import functools
import math

import numpy as np
import jax
import jax.numpy as jnp
from jax import lax
from jax.experimental import pallas as pl
from jax.experimental.pallas import tpu as pltpu
from jax.experimental.pallas import tpu_sc as plsc

D_MODEL = 768
MAX_SEQ_LEN = 2048
_SCALE = math.sqrt(float(D_MODEL))
_LANES = 16


def _pe_host() -> np.ndarray:
    pos = np.arange(MAX_SEQ_LEN, dtype=np.float64).reshape(-1, 1)
    i = np.arange(D_MODEL, dtype=np.float64)
    rads = pos / np.power(10000.0, 2.0 * np.floor(i / 2.0) / D_MODEL)
    pe = np.zeros((MAX_SEQ_LEN, D_MODEL), dtype=np.float32)
    pe[:, 0::2] = np.sin(rads[:, 0::2]).astype(np.float32)
    pe[:, 1::2] = np.cos(rads[:, 1::2]).astype(np.float32)
    return pe


_PE = _pe_host()


@functools.lru_cache(maxsize=None)
def _build(batch: int, seq: int):
    info = plsc.get_sparse_core_info()
    nc, ns = info.num_cores, info.num_subcores
    nw = nc * ns
    ppw = seq // nw
    pchunk = 32
    nchunk = ppw // pchunk
    groups = D_MODEL // _LANES

    mesh = plsc.VectorSubcoreMesh(core_axis_name="c", subcore_axis_name="s")

    @functools.partial(
        pl.kernel,
        mesh=mesh,
        out_type=jax.ShapeDtypeStruct((batch, seq, D_MODEL), jnp.float32),
        scratch_types=[
            pltpu.VMEM((batch, ppw), jnp.int32),
            pltpu.VMEM((batch * pchunk, D_MODEL), jnp.float32),
            pltpu.VMEM((pchunk, D_MODEL), jnp.float32),
            pltpu.SemaphoreType.DMA,
            pltpu.SemaphoreType.DMA,
            pltpu.SemaphoreType.DMA,
        ],
    )
    def emb(x_hbm, table_hbm, pe_hbm, out_hbm, idx_v, tok_v, pe_v,
            sem_g, sem_p, sem_o):
        wid = lax.axis_index("s") * nc + lax.axis_index("c")
        pos0 = wid * ppw
        for b in range(batch):
            pltpu.sync_copy(x_hbm.at[b, pl.ds(pos0, ppw)], idx_v.at[b])
        for c in range(nchunk):
            pos = pos0 + c * pchunk
            gs = [
                pltpu.async_copy(
                    table_hbm.at[idx_v.at[b, pl.ds(c * pchunk, pchunk)]],
                    tok_v.at[pl.ds(b * pchunk, pchunk)], sem_g)
                for b in range(batch)
            ]
            p = pltpu.async_copy(pe_hbm.at[pl.ds(pos, pchunk)], pe_v, sem_p)
            for g in gs:
                g.wait()
            p.wait()

            @plsc.parallel_loop(0, pchunk, unroll=2)
            def _row(r):
                for gi in range(groups):
                    sl = pl.ds(gi * _LANES, _LANES)
                    vp = pe_v[r, sl]
                    for b in range(batch):
                        tok_v[b * pchunk + r, sl] = (
                            tok_v[b * pchunk + r, sl] * _SCALE + vp)

            outs = [
                pltpu.async_copy(
                    tok_v.at[pl.ds(b * pchunk, pchunk)],
                    out_hbm.at[b, pl.ds(pos, pchunk)], sem_o)
                for b in range(batch)
            ]
            for o in outs:
                o.wait()

    return emb


def kernel(x, table):
    b, s = x.shape
    emb = _build(b, s)
    pe = jnp.asarray(_PE)
    return emb(x, table, pe)


# speedup vs baseline: 1.8050x; 1.8050x over previous
import functools
import math

import numpy as np
import jax
import jax.numpy as jnp
from jax import lax
from jax.experimental import pallas as pl
from jax.experimental.pallas import tpu as pltpu
from jax.experimental.pallas import tpu_sc as plsc

D_MODEL = 768
MAX_SEQ_LEN = 2048
_SCALE = math.sqrt(float(D_MODEL))
_LANES = 16


def _pe_host() -> np.ndarray:
    pos = np.arange(MAX_SEQ_LEN, dtype=np.float64).reshape(-1, 1)
    i = np.arange(D_MODEL, dtype=np.float64)
    rads = pos / np.power(10000.0, 2.0 * np.floor(i / 2.0) / D_MODEL)
    pe = np.zeros((MAX_SEQ_LEN, D_MODEL), dtype=np.float32)
    pe[:, 0::2] = np.sin(rads[:, 0::2]).astype(np.float32)
    pe[:, 1::2] = np.cos(rads[:, 1::2]).astype(np.float32)
    return pe


_PE = _pe_host()


@functools.lru_cache(maxsize=None)
def _build(batch: int, seq: int):
    info = plsc.get_sparse_core_info()
    nc, ns = info.num_cores, info.num_subcores
    nw = nc * ns
    rpw = batch * seq // nw
    wpb = nw // batch
    chunk = 64
    nchunk = rpw // chunk
    groups = D_MODEL // _LANES

    mesh = plsc.VectorSubcoreMesh(core_axis_name="c", subcore_axis_name="s")

    @functools.partial(
        pl.kernel,
        mesh=mesh,
        out_type=jax.ShapeDtypeStruct((batch, seq, D_MODEL), jnp.float32),
        scratch_types=[
            pltpu.VMEM((rpw,), jnp.int32),
            pltpu.VMEM((chunk, D_MODEL), jnp.float32),
            pltpu.VMEM((chunk, D_MODEL), jnp.float32),
            pltpu.SemaphoreType.DMA,
            pltpu.SemaphoreType.DMA,
            pltpu.SemaphoreType.DMA,
        ],
    )
    def emb(x_hbm, table_hbm, pe_hbm, out_hbm, idx_v, tok_v, pe_v,
            sem_g, sem_p, sem_o):
        wid = lax.axis_index("s") * nc + lax.axis_index("c")
        bi = wid // wpb
        seq0 = (wid % wpb) * rpw
        pltpu.sync_copy(x_hbm.at[bi, pl.ds(seq0, rpw)], idx_v)
        for c in range(nchunk):
            g = pltpu.async_copy(
                table_hbm.at[idx_v.at[pl.ds(c * chunk, chunk)]], tok_v, sem_g)
            p = pltpu.async_copy(
                pe_hbm.at[pl.ds(seq0 + c * chunk, chunk)], pe_v, sem_p)
            g.wait()
            p.wait()

            @plsc.parallel_loop(0, chunk, unroll=4)
            def _row(r):
                for gi in range(groups):
                    sl = pl.ds(gi * _LANES, _LANES)
                    tok_v[r, sl] = tok_v[r, sl] * _SCALE + pe_v[r, sl]

            pltpu.async_copy(
                tok_v, out_hbm.at[bi, pl.ds(seq0 + c * chunk, chunk)],
                sem_o).wait()

    return emb


def kernel(x, table):
    b, s = x.shape
    emb = _build(b, s)
    pe = jnp.asarray(_PE)
    return emb(x, table, pe)


# speedup vs baseline: 2.1857x; 1.2109x over previous
import functools
import math

import numpy as np
import jax
import jax.numpy as jnp
from jax import lax
from jax.experimental import pallas as pl
from jax.experimental.pallas import tpu as pltpu
from jax.experimental.pallas import tpu_sc as plsc

D_MODEL = 768
MAX_SEQ_LEN = 2048
_SCALE = math.sqrt(float(D_MODEL))
_LANES = 16


def _pe_host() -> np.ndarray:
    pos = np.arange(MAX_SEQ_LEN, dtype=np.float64).reshape(-1, 1)
    i = np.arange(D_MODEL, dtype=np.float64)
    rads = pos / np.power(10000.0, 2.0 * np.floor(i / 2.0) / D_MODEL)
    pe = np.zeros((MAX_SEQ_LEN, D_MODEL), dtype=np.float32)
    pe[:, 0::2] = np.sin(rads[:, 0::2]).astype(np.float32)
    pe[:, 1::2] = np.cos(rads[:, 1::2]).astype(np.float32)
    return pe


_PE = _pe_host()


@functools.lru_cache(maxsize=None)
def _build(batch: int, seq: int):
    info = plsc.get_sparse_core_info()
    nc, ns = info.num_cores, info.num_subcores
    nw = nc * ns
    rpw = batch * seq // nw
    wpb = nw // batch
    chunk = 32
    nchunk = rpw // chunk
    groups = D_MODEL // _LANES

    mesh = plsc.VectorSubcoreMesh(core_axis_name="c", subcore_axis_name="s")

    @functools.partial(
        pl.kernel,
        mesh=mesh,
        out_type=jax.ShapeDtypeStruct((batch, seq, D_MODEL), jnp.float32),
        scratch_types=[
            pltpu.VMEM((rpw,), jnp.int32),
            pltpu.VMEM((2, chunk, D_MODEL), jnp.float32),
            pltpu.VMEM((2, chunk, D_MODEL), jnp.float32),
            pltpu.SemaphoreType.DMA,
            pltpu.SemaphoreType.DMA,
            pltpu.SemaphoreType.DMA,
            pltpu.SemaphoreType.DMA,
            pltpu.SemaphoreType.DMA,
            pltpu.SemaphoreType.DMA,
        ],
    )
    def emb(x_hbm, table_hbm, pe_hbm, out_hbm, idx_v, tok_v, pe_v,
            sg0, sg1, sp0, sp1, so0, so1):
        sg, sp, so = (sg0, sg1), (sp0, sp1), (so0, so1)
        wid = lax.axis_index("s") * nc + lax.axis_index("c")
        bi = wid // wpb
        seq0 = (wid % wpb) * rpw
        pltpu.sync_copy(x_hbm.at[bi, pl.ds(seq0, rpw)], idx_v)

        def start_in(c):
            b = c & 1
            return (
                pltpu.async_copy(
                    table_hbm.at[idx_v.at[pl.ds(c * chunk, chunk)]],
                    tok_v.at[b], sg[b]),
                pltpu.async_copy(
                    pe_hbm.at[pl.ds(seq0 + c * chunk, chunk)],
                    pe_v.at[b], sp[b]),
            )

        pend_in = {0: start_in(0)}
        pend_out = {}
        for c in range(nchunk):
            b = c & 1
            if c + 1 < nchunk:
                if c - 1 in pend_out:
                    pend_out.pop(c - 1).wait()
                pend_in[c + 1] = start_in(c + 1)
            g, p = pend_in.pop(c)
            g.wait()
            p.wait()

            @plsc.parallel_loop(0, chunk, unroll=2)
            def _row(r, b=b):
                for gi in range(groups):
                    sl = pl.ds(gi * _LANES, _LANES)
                    tok_v[b, r, sl] = tok_v[b, r, sl] * _SCALE + pe_v[b, r, sl]

            pend_out[c] = pltpu.async_copy(
                tok_v.at[b], out_hbm.at[bi, pl.ds(seq0 + c * chunk, chunk)],
                so[b])
        for c in sorted(pend_out):
            pend_out.pop(c).wait()

    return emb


def kernel(x, table):
    b, s = x.shape
    emb = _build(b, s)
    pe = jnp.asarray(_PE)
    return emb(x, table, pe)
